# baseline (speedup 1.0000x reference)
import functools

import jax
import jax.numpy as jnp
from jax import lax
from jax.experimental import pallas as pl
from jax.experimental.pallas import tpu as pltpu
from jax.experimental.pallas import tpu_sc as plsc

B = 1024
D = 64
CAP = 100000
K = 32
CBLK = 1024
CPAD = 100352
NTILES = CPAD // CBLK
CHW = 64
NCH = CBLK // CHW
EPS = 1e-8
NEG = float("-inf")
IMAX = 2**31 - 1


def _topk_body(q_ref, wdg_ref, keys_ref, blk_ref, half_ref, nov_ref,
               keyn_s, topv_s, topi_s, sim_s, cmax_s, wm_s):
    c = pl.program_id(0)

    @pl.when(c == 0)
    def _init():
        x = lax.dot_general(q_ref[...], wdg_ref[...],
                            (((1,), (1,)), ((), ())),
                            preferred_element_type=jnp.float32)
        ii = lax.broadcasted_iota(jnp.int32, (B, D), 1)
        rank = jnp.zeros((B, D), jnp.int32)
        for j in range(D):
            xj = x[:, j:j + 1]
            rank += ((xj > x) | ((xj == x) & (ii > j))).astype(jnp.int32)
        xm = jnp.where(rank < K, x, 0.0)
        n2 = jnp.sum(xm * xm, axis=1, keepdims=True)
        keyn_s[...] = xm / jnp.maximum(jnp.sqrt(n2), EPS)
        topv_s[...] = jnp.full((B, K), NEG, jnp.float32)
        topi_s[...] = jnp.zeros((B, K), jnp.int32)

    kt = keys_ref[...]
    n2 = jnp.sum(kt * kt, axis=1, keepdims=True)
    ktn = kt / jnp.maximum(jnp.sqrt(n2), EPS)
    sim = lax.dot_general(keyn_s[...], ktn,
                          (((1,), (1,)), ((), ())),
                          preferred_element_type=jnp.float32)
    gidx = c * CBLK + lax.broadcasted_iota(jnp.int32, (B, CBLK), 1)
    sim = jnp.where(gidx < CAP, sim, NEG)

    lane_ch = lax.broadcasted_iota(jnp.int32, (B, NCH), 1)
    cmax = jnp.full((B, NCH), NEG, jnp.float32)
    for k in range(NCH):
        sl = sim[:, k * CHW:(k + 1) * CHW]
        sim_s[:, k * CHW:(k + 1) * CHW] = sl
        cm = jnp.max(sl, axis=1, keepdims=True)
        cmax = jnp.where(lane_ch == k, cm, cmax)

    lane_c = lax.broadcasted_iota(jnp.int32, (B, CHW), 1)
    lane_k = lax.broadcasted_iota(jnp.int32, (B, K), 1)

    def _round(j, carry):
        ov, oi, cm, wm = carry
        tt = jnp.min(ov, axis=1, keepdims=True)
        m = jnp.max(cm, axis=1, keepdims=True)
        upd = m > tt
        chsel = jnp.min(jnp.where(cm == m, lane_ch, NCH), axis=1,
                        keepdims=True)
        wmsel = jnp.sum(jnp.where(lane_ch == chsel, wm, 0), axis=1,
                        keepdims=True)
        chv = jnp.full((B, CHW), NEG, jnp.float32)
        for k in range(NCH):
            chv = jnp.where(chsel == k, sim_s[:, k * CHW:(k + 1) * CHW],
                            chv)
        elig = (chv == m) & (lane_c >= wmsel)
        lsel = jnp.min(jnp.where(elig, lane_c, CHW), axis=1, keepdims=True)
        gsel = c * CBLK + chsel * CHW + lsel
        eqlater = (chv == m) & (lane_c > lsel)
        below = jnp.where(chv < m, chv, NEG)
        nm = jnp.max(jnp.where(eqlater, chv, below), axis=1, keepdims=True)
        nwm = jnp.where(nm == m, lsel + 1, 0)
        hitch = (lane_ch == chsel) & upd
        cm = jnp.where(hitch, nm, cm)
        wm = jnp.where(hitch, nwm, wm)
        minpos = jnp.min(jnp.where(ov == tt, lane_k, K), axis=1,
                         keepdims=True)
        colsel = lane_k == minpos
        ov = jnp.where(upd & colsel, m, ov)
        oi = jnp.where(upd & colsel, gsel, oi)
        return ov, oi, cm, wm

    wm0 = jnp.zeros((B, NCH), jnp.int32)
    ov, oi, cm, wm = lax.fori_loop(
        0, 8, _round, (topv_s[...], topi_s[...], cmax, wm0))
    topv_s[...] = ov
    topi_s[...] = oi
    cmax_s[...] = cm
    wm_s[...] = wm
    need = jnp.sum((jnp.max(cm, axis=1, keepdims=True) >
                    jnp.min(ov, axis=1, keepdims=True)).astype(jnp.int32),
                   axis=(0, 1), keepdims=True)

    @pl.when(need[0, 0] > 0)
    def _phase2():
        ov2, oi2, cm2, wm2 = lax.fori_loop(
            0, K - 8, _round,
            (topv_s[...], topi_s[...], cmax_s[...], wm_s[...]))
        topv_s[...] = ov2
        topi_s[...] = oi2

    @pl.when(c == NTILES - 1)
    def _fin():
        ov = topv_s[...]
        oi = topi_s[...]
        rank = jnp.zeros((B, K), jnp.int32)
        for j in range(K):
            vj = ov[:, j:j + 1]
            ij = oi[:, j:j + 1]
            rank += ((vj > ov) | ((vj == ov) & (ij < oi))).astype(jnp.int32)
        si = jnp.zeros((B, K), jnp.int32)
        for r in range(K):
            selr = rank == r
            ir = jnp.sum(jnp.where(selr, oi, 0), axis=1, keepdims=True)
            si = jnp.where(lane_k == r, ir, si)
        blk_ref[...] = si >> 1
        half_ref[...] = si & 1
        mx = jnp.max(ov, axis=1, keepdims=True)
        nov_ref[...] = jnp.clip(1.0 - jnp.clip(mx, -1.0, 1.0), 0.0, 1.0)


def _topk_call(query, keys_pad, w_dg):
    return pl.pallas_call(
        _topk_body,
        grid=(NTILES,),
        in_specs=[
            pl.BlockSpec((B, D), lambda c: (0, 0)),
            pl.BlockSpec((D, D), lambda c: (0, 0)),
            pl.BlockSpec((CBLK, D), lambda c: (c, 0)),
        ],
        out_specs=[
            pl.BlockSpec((B, K), lambda c: (0, 0)),
            pl.BlockSpec((B, K), lambda c: (0, 0)),
            pl.BlockSpec((B, 1), lambda c: (0, 0)),
        ],
        out_shape=[
            jax.ShapeDtypeStruct((B, K), jnp.int32),
            jax.ShapeDtypeStruct((B, K), jnp.int32),
            jax.ShapeDtypeStruct((B, 1), jnp.float32),
        ],
        scratch_shapes=[
            pltpu.VMEM((B, D), jnp.float32),
            pltpu.VMEM((B, K), jnp.float32),
            pltpu.VMEM((B, K), jnp.int32),
            pltpu.VMEM((B, CBLK), jnp.float32),
            pltpu.VMEM((B, NCH), jnp.float32),
            pltpu.VMEM((B, NCH), jnp.int32),
        ],
        compiler_params=pltpu.CompilerParams(
            dimension_semantics=("arbitrary",)),
    )(query, w_dg, keys_pad)


_NC = 2
_NS = 16
_NW = _NC * _NS
_BPW = (B * K) // _NW
_CHUNK = 128
_NCHUNK = _BPW // _CHUNK


def _gather_call(blk_idx, table128):
    mesh = plsc.VectorSubcoreMesh(core_axis_name="c", subcore_axis_name="s")

    @functools.partial(
        pl.kernel,
        mesh=mesh,
        out_type=jax.ShapeDtypeStruct((B * K, 2 * D), jnp.float32),
        scratch_types=[
            pltpu.VMEM((_NCHUNK, _CHUNK), jnp.int32),
            pltpu.VMEM((2, _CHUNK, 2 * D), jnp.float32),
            pltpu.SemaphoreType.DMA,
        ],
    )
    def _gather(idx_hbm, table_hbm, out_hbm, idx_v, bufs, sem):
        wid = lax.axis_index("s") * _NC + lax.axis_index("c")
        base = wid * _BPW
        pltpu.sync_copy(idx_hbm.at[pl.ds(wid * _NCHUNK, _NCHUNK)], idx_v)
        handles = []
        for j in range(_NCHUNK):
            handles.append(
                pltpu.async_copy(table_hbm.at[idx_v.at[j]],
                                 bufs.at[j % 2], sem))
            if j > 0:
                handles[j - 1].wait()
                pltpu.sync_copy(
                    bufs.at[(j - 1) % 2],
                    out_hbm.at[pl.ds(base + (j - 1) * _CHUNK, _CHUNK)])
        handles[-1].wait()
        pltpu.sync_copy(
            bufs.at[(_NCHUNK - 1) % 2],
            out_hbm.at[pl.ds(base + (_NCHUNK - 1) * _CHUNK, _CHUNK)])

    return _gather(blk_idx, table128)


def _ca1_body(g_ref, h_ref, w_ref, o_ref):
    g = g_ref[...]
    h = h_ref[...]
    lo = g[:, :D]
    hi = g[:, D:]
    rows = jnp.where(h == 0, lo, hi)
    o_ref[...] = lax.dot_general(rows, w_ref[...],
                                 (((1,), (1,)), ((), ())),
                                 preferred_element_type=jnp.float32)


def _ca1_call(g, half, w_ca1):
    rows = B * K
    blk = 8192
    return pl.pallas_call(
        _ca1_body,
        grid=(rows // blk,),
        in_specs=[
            pl.BlockSpec((blk, 2 * D), lambda i: (i, 0)),
            pl.BlockSpec((blk, 1), lambda i: (i, 0)),
            pl.BlockSpec((D, D), lambda i: (0, 0)),
        ],
        out_specs=pl.BlockSpec((blk, D), lambda i: (i, 0)),
        out_shape=jax.ShapeDtypeStruct((rows, D), jnp.float32),
    )(g, half, w_ca1)


def kernel(query, keys_mem, values_mem, W_dg, W_ca1):
    keys_pad = jnp.pad(keys_mem, ((0, CPAD - CAP), (0, 0)))
    blk, half, nov = _topk_call(query, keys_pad, W_dg)
    table128 = values_mem.reshape(CAP // 2, 2 * D)
    gathered = _gather_call(blk.reshape(-1, _CHUNK), table128)
    recalled = _ca1_call(gathered, half.reshape(-1, 1), W_ca1)
    return recalled.reshape(B, K, D), nov.reshape(B)


# speedup vs baseline: 9.3980x; 9.3980x over previous
import functools

import jax
import jax.numpy as jnp
from jax import lax
from jax.experimental import pallas as pl
from jax.experimental.pallas import tpu as pltpu
from jax.experimental.pallas import tpu_sc as plsc

B = 1024
D = 64
CAP = 100000
K = 32
CBLK = 1024
CPAD = 100352
NTILES = CPAD // CBLK
CSH = 32
NCH = CBLK // CSH
P1 = 8
EPS = 1e-8
NEG = float("-inf")


def _topk_body(q_ref, wdg_ref, keys_ref, blk_ref, half_ref, nov_ref,
               keyn_s, topv_s, topi_s, sim_s, cmax_s, wm_s):
    c = pl.program_id(0)

    @pl.when(c == 0)
    def _init():
        x = lax.dot_general(q_ref[...], wdg_ref[...],
                            (((1,), (1,)), ((), ())),
                            preferred_element_type=jnp.float32)
        ii = lax.broadcasted_iota(jnp.int32, (B, D), 1)
        rank = jnp.zeros((B, D), jnp.int32)
        for j in range(D):
            xj = x[:, j:j + 1]
            rank += ((xj > x) | ((xj == x) & (ii > j))).astype(jnp.int32)
        xm = jnp.where(rank < K, x, 0.0)
        n2 = jnp.sum(xm * xm, axis=1, keepdims=True)
        keyn_s[...] = xm / jnp.maximum(jnp.sqrt(n2), EPS)
        topv_s[...] = jnp.full((K, B), NEG, jnp.float32)
        topi_s[...] = jnp.zeros((K, B), jnp.int32)

    kt = keys_ref[...]
    n2 = jnp.sum(kt * kt, axis=1, keepdims=True)
    ktn = kt / jnp.maximum(jnp.sqrt(n2), EPS)
    simt = lax.dot_general(ktn, keyn_s[...],
                           (((1,), (1,)), ((), ())),
                           preferred_element_type=jnp.float32)
    ridx = c * CBLK + lax.broadcasted_iota(jnp.int32, (CBLK, B), 0)
    simt = jnp.where(ridx < CAP, simt, NEG)

    sub_ch = lax.broadcasted_iota(jnp.int32, (NCH, B), 0)
    cmax = jnp.full((NCH, B), NEG, jnp.float32)
    for k in range(NCH):
        sl = simt[k * CSH:(k + 1) * CSH, :]
        sim_s[k * CSH:(k + 1) * CSH, :] = sl
        cm = jnp.max(sl, axis=0, keepdims=True)
        cmax = jnp.where(sub_ch == k, cm, cmax)

    sub_c = lax.broadcasted_iota(jnp.int32, (CSH, B), 0)
    sub_k = lax.broadcasted_iota(jnp.int32, (K, B), 0)

    def _round(j, carry):
        ov, oi, cm, wm = carry
        tt = jnp.min(ov, axis=0, keepdims=True)
        m = jnp.max(cm, axis=0, keepdims=True)
        upd = m > tt
        chsel = jnp.min(jnp.where(cm == m, sub_ch, NCH), axis=0,
                        keepdims=True)
        wmsel = jnp.sum(jnp.where(sub_ch == chsel, wm, 0), axis=0,
                        keepdims=True)
        chv = jnp.full((CSH, B), NEG, jnp.float32)
        for k in range(NCH):
            chv = jnp.where(chsel == k, sim_s[k * CSH:(k + 1) * CSH, :],
                            chv)
        elig = (chv == m) & (sub_c >= wmsel)
        lsel = jnp.min(jnp.where(elig, sub_c, CSH), axis=0, keepdims=True)
        gsel = c * CBLK + chsel * CSH + lsel
        eqlater = (chv == m) & (sub_c > lsel)
        below = jnp.where(chv < m, chv, NEG)
        nm = jnp.max(jnp.where(eqlater, chv, below), axis=0, keepdims=True)
        nwm = jnp.where(nm == m, lsel + 1, 0)
        hitch = (sub_ch == chsel) & upd
        cm = jnp.where(hitch, nm, cm)
        wm = jnp.where(hitch, nwm, wm)
        minpos = jnp.min(jnp.where(ov == tt, sub_k, K), axis=0,
                         keepdims=True)
        colsel = sub_k == minpos
        ov = jnp.where(upd & colsel, m, ov)
        oi = jnp.where(upd & colsel, gsel, oi)
        return ov, oi, cm, wm

    wm0 = jnp.zeros((NCH, B), jnp.int32)
    ov, oi, cm, wm = lax.fori_loop(
        0, P1, _round, (topv_s[...], topi_s[...], cmax, wm0))
    topv_s[...] = ov
    topi_s[...] = oi
    cmax_s[...] = cm
    wm_s[...] = wm
    need = jnp.sum((jnp.max(cm, axis=0, keepdims=True) >
                    jnp.min(ov, axis=0, keepdims=True)).astype(jnp.int32),
                   axis=(0, 1), keepdims=True)

    @pl.when(need[0, 0] > 0)
    def _phase2():
        ov2, oi2, _, _ = lax.fori_loop(
            0, K - P1, _round,
            (topv_s[...], topi_s[...], cmax_s[...], wm_s[...]))
        topv_s[...] = ov2
        topi_s[...] = oi2

    @pl.when(c == NTILES - 1)
    def _fin():
        ov = topv_s[...]
        oi = topi_s[...]
        rank = jnp.zeros((K, B), jnp.int32)
        for j in range(K):
            vj = ov[j:j + 1, :]
            ij = oi[j:j + 1, :]
            rank += ((vj > ov) | ((vj == ov) & (ij < oi))).astype(jnp.int32)
        si = jnp.zeros((K, B), jnp.int32)
        for r in range(K):
            selr = rank == r
            ir = jnp.sum(jnp.where(selr, oi, 0), axis=0, keepdims=True)
            si = jnp.where(sub_k == r, ir, si)
        blk_ref[...] = si >> 1
        half_ref[...] = si & 1
        mx = jnp.max(ov, axis=0, keepdims=True)
        nov_ref[...] = jnp.clip(1.0 - jnp.clip(mx, -1.0, 1.0), 0.0, 1.0)


def _topk_call(query, keys_pad, w_dg):
    return pl.pallas_call(
        _topk_body,
        grid=(NTILES,),
        in_specs=[
            pl.BlockSpec((B, D), lambda c: (0, 0)),
            pl.BlockSpec((D, D), lambda c: (0, 0)),
            pl.BlockSpec((CBLK, D), lambda c: (c, 0)),
        ],
        out_specs=[
            pl.BlockSpec((K, B), lambda c: (0, 0)),
            pl.BlockSpec((K, B), lambda c: (0, 0)),
            pl.BlockSpec((1, B), lambda c: (0, 0)),
        ],
        out_shape=[
            jax.ShapeDtypeStruct((K, B), jnp.int32),
            jax.ShapeDtypeStruct((K, B), jnp.int32),
            jax.ShapeDtypeStruct((1, B), jnp.float32),
        ],
        scratch_shapes=[
            pltpu.VMEM((B, D), jnp.float32),
            pltpu.VMEM((K, B), jnp.float32),
            pltpu.VMEM((K, B), jnp.int32),
            pltpu.VMEM((CBLK, B), jnp.float32),
            pltpu.VMEM((NCH, B), jnp.float32),
            pltpu.VMEM((NCH, B), jnp.int32),
        ],
        compiler_params=pltpu.CompilerParams(
            dimension_semantics=("arbitrary",)),
    )(query, w_dg, keys_pad)


_NC = 2
_NS = 16
_NW = _NC * _NS
_BPW = (B * K) // _NW
_CHUNK = 128
_NCHUNK = _BPW // _CHUNK


def _gather_call(blk_idx, table128):
    mesh = plsc.VectorSubcoreMesh(core_axis_name="c", subcore_axis_name="s")

    @functools.partial(
        pl.kernel,
        mesh=mesh,
        out_type=jax.ShapeDtypeStruct((B * K, 2 * D), jnp.float32),
        scratch_types=[
            pltpu.VMEM((_NCHUNK, _CHUNK), jnp.int32),
            pltpu.VMEM((2, _CHUNK, 2 * D), jnp.float32),
            pltpu.SemaphoreType.DMA,
        ],
    )
    def _gather(idx_hbm, table_hbm, out_hbm, idx_v, bufs, sem):
        wid = lax.axis_index("s") * _NC + lax.axis_index("c")
        base = wid * _BPW
        pltpu.sync_copy(idx_hbm.at[pl.ds(wid * _NCHUNK, _NCHUNK)], idx_v)
        handles = []
        for j in range(_NCHUNK):
            handles.append(
                pltpu.async_copy(table_hbm.at[idx_v.at[j]],
                                 bufs.at[j % 2], sem))
            if j > 0:
                handles[j - 1].wait()
                pltpu.sync_copy(
                    bufs.at[(j - 1) % 2],
                    out_hbm.at[pl.ds(base + (j - 1) * _CHUNK, _CHUNK)])
        handles[-1].wait()
        pltpu.sync_copy(
            bufs.at[(_NCHUNK - 1) % 2],
            out_hbm.at[pl.ds(base + (_NCHUNK - 1) * _CHUNK, _CHUNK)])

    return _gather(blk_idx, table128)


def _ca1_body(g_ref, h_ref, w_ref, o_ref):
    g = g_ref[...]
    h = h_ref[...]
    lo = g[:, :D]
    hi = g[:, D:]
    rows = jnp.where(h == 0, lo, hi)
    o_ref[...] = lax.dot_general(rows, w_ref[...],
                                 (((1,), (1,)), ((), ())),
                                 preferred_element_type=jnp.float32)


def _ca1_call(g, half, w_ca1):
    rows = B * K
    blk = 8192
    return pl.pallas_call(
        _ca1_body,
        grid=(rows // blk,),
        in_specs=[
            pl.BlockSpec((blk, 2 * D), lambda i: (i, 0)),
            pl.BlockSpec((blk, 1), lambda i: (i, 0)),
            pl.BlockSpec((D, D), lambda i: (0, 0)),
        ],
        out_specs=pl.BlockSpec((blk, D), lambda i: (i, 0)),
        out_shape=jax.ShapeDtypeStruct((rows, D), jnp.float32),
    )(g, half, w_ca1)


def kernel(query, keys_mem, values_mem, W_dg, W_ca1):
    keys_pad = jnp.pad(keys_mem, ((0, CPAD - CAP), (0, 0)))
    blk_t, half_t, nov = _topk_call(query, keys_pad, W_dg)
    blk = blk_t.T
    half = half_t.T
    table128 = values_mem.reshape(CAP // 2, 2 * D)
    gathered = _gather_call(blk.reshape(-1, _CHUNK), table128)
    recalled = _ca1_call(gathered, half.reshape(-1, 1), W_ca1)
    return recalled.reshape(B, K, D), nov.reshape(B)


# speedup vs baseline: 9.6885x; 1.0309x over previous
import functools

import jax
import jax.numpy as jnp
from jax import lax
from jax.experimental import pallas as pl
from jax.experimental.pallas import tpu as pltpu
from jax.experimental.pallas import tpu_sc as plsc

B = 1024
D = 64
CAP = 100000
K = 32
CBLK = 1024
CPAD = 100352
NTILES = CPAD // CBLK
CSH = 32
NCH = CBLK // CSH
P1 = 8
EPS = 1e-8
NEG = float("-inf")


def _topk_body(q_ref, wdg_ref, keys_ref, blk_ref, half_ref, nov_ref,
               keyn_s, topv_s, topi_s, sim_s, cmax_s, wm_s):
    c = pl.program_id(0)

    @pl.when(c == 0)
    def _init():
        x = lax.dot_general(q_ref[...], wdg_ref[...],
                            (((1,), (1,)), ((), ())),
                            preferred_element_type=jnp.float32)
        ii = lax.broadcasted_iota(jnp.int32, (B, D), 1)
        rank = jnp.zeros((B, D), jnp.int32)
        for j in range(D):
            xj = x[:, j:j + 1]
            rank += ((xj > x) | ((xj == x) & (ii > j))).astype(jnp.int32)
        xm = jnp.where(rank < K, x, 0.0)
        n2 = jnp.sum(xm * xm, axis=1, keepdims=True)
        keyn_s[...] = xm / jnp.maximum(jnp.sqrt(n2), EPS)
        topv_s[...] = jnp.full((K, B), NEG, jnp.float32)
        topi_s[...] = jnp.zeros((K, B), jnp.int32)

    kt = keys_ref[...]
    n2 = jnp.sum(kt * kt, axis=1, keepdims=True)
    ktn = kt / jnp.maximum(jnp.sqrt(n2), EPS)
    simt = lax.dot_general(ktn, keyn_s[...],
                           (((1,), (1,)), ((), ())),
                           preferred_element_type=jnp.float32)
    ridx = c * CBLK + lax.broadcasted_iota(jnp.int32, (CBLK, B), 0)
    simt = jnp.where(ridx < CAP, simt, NEG)

    sub_ch = lax.broadcasted_iota(jnp.int32, (NCH, B), 0)
    cmax = jnp.full((NCH, B), NEG, jnp.float32)
    for k in range(NCH):
        sl = simt[k * CSH:(k + 1) * CSH, :]
        sim_s[k * CSH:(k + 1) * CSH, :] = sl
        cm = jnp.max(sl, axis=0, keepdims=True)
        cmax = jnp.where(sub_ch == k, cm, cmax)

    sub_c = lax.broadcasted_iota(jnp.int32, (CSH, B), 0)
    sub_k = lax.broadcasted_iota(jnp.int32, (K, B), 0)

    def _round(j, carry):
        ov, oi, cm, wm = carry
        tt = jnp.min(ov, axis=0, keepdims=True)
        m = jnp.max(cm, axis=0, keepdims=True)
        upd = m > tt
        chsel = jnp.min(jnp.where(cm == m, sub_ch, NCH), axis=0,
                        keepdims=True)
        wmsel = jnp.sum(jnp.where(sub_ch == chsel, wm, 0), axis=0,
                        keepdims=True)
        chv = jnp.full((CSH, B), NEG, jnp.float32)
        for k in range(NCH):
            chv = jnp.where(chsel == k, sim_s[k * CSH:(k + 1) * CSH, :],
                            chv)
        elig = (chv == m) & (sub_c >= wmsel)
        lsel = jnp.min(jnp.where(elig, sub_c, CSH), axis=0, keepdims=True)
        gsel = c * CBLK + chsel * CSH + lsel
        eqlater = (chv == m) & (sub_c > lsel)
        below = jnp.where(chv < m, chv, NEG)
        nm = jnp.max(jnp.where(eqlater, chv, below), axis=0, keepdims=True)
        nwm = jnp.where(nm == m, lsel + 1, 0)
        hitch = (sub_ch == chsel) & upd
        cm = jnp.where(hitch, nm, cm)
        wm = jnp.where(hitch, nwm, wm)
        minpos = jnp.min(jnp.where(ov == tt, sub_k, K), axis=0,
                         keepdims=True)
        colsel = sub_k == minpos
        ov = jnp.where(upd & colsel, m, ov)
        oi = jnp.where(upd & colsel, gsel, oi)
        return ov, oi, cm, wm

    wm0 = jnp.zeros((NCH, B), jnp.int32)
    ov, oi, cm, wm = lax.fori_loop(
        0, P1, _round, (topv_s[...], topi_s[...], cmax, wm0))
    topv_s[...] = ov
    topi_s[...] = oi
    cmax_s[...] = cm
    wm_s[...] = wm
    need = jnp.sum((jnp.max(cm, axis=0, keepdims=True) >
                    jnp.min(ov, axis=0, keepdims=True)).astype(jnp.int32),
                   axis=(0, 1), keepdims=True)

    @pl.when(need[0, 0] > 0)
    def _phase2():
        ov2, oi2, _, _ = lax.fori_loop(
            0, K - P1, _round,
            (topv_s[...], topi_s[...], cmax_s[...], wm_s[...]))
        topv_s[...] = ov2
        topi_s[...] = oi2

    @pl.when(c == NTILES - 1)
    def _fin():
        ov = topv_s[...]
        oi = topi_s[...]
        rank = jnp.zeros((K, B), jnp.int32)
        for j in range(K):
            vj = ov[j:j + 1, :]
            ij = oi[j:j + 1, :]
            rank += ((vj > ov) | ((vj == ov) & (ij < oi))).astype(jnp.int32)
        si = jnp.zeros((K, B), jnp.int32)
        for r in range(K):
            selr = rank == r
            ir = jnp.sum(jnp.where(selr, oi, 0), axis=0, keepdims=True)
            si = jnp.where(sub_k == r, ir, si)
        sit = si.T
        blk_ref[...] = sit >> 1
        half_ref[...] = sit & 1
        mx = jnp.max(ov, axis=0, keepdims=True)
        nov_ref[...] = jnp.clip(1.0 - jnp.clip(mx, -1.0, 1.0), 0.0, 1.0)


def _topk_call(query, keys_pad, w_dg):
    return pl.pallas_call(
        _topk_body,
        grid=(NTILES,),
        in_specs=[
            pl.BlockSpec((B, D), lambda c: (0, 0)),
            pl.BlockSpec((D, D), lambda c: (0, 0)),
            pl.BlockSpec((CBLK, D), lambda c: (c, 0)),
        ],
        out_specs=[
            pl.BlockSpec((B, K), lambda c: (0, 0)),
            pl.BlockSpec((B, K), lambda c: (0, 0)),
            pl.BlockSpec((1, B), lambda c: (0, 0)),
        ],
        out_shape=[
            jax.ShapeDtypeStruct((B, K), jnp.int32),
            jax.ShapeDtypeStruct((B, K), jnp.int32),
            jax.ShapeDtypeStruct((1, B), jnp.float32),
        ],
        scratch_shapes=[
            pltpu.VMEM((B, D), jnp.float32),
            pltpu.VMEM((K, B), jnp.float32),
            pltpu.VMEM((K, B), jnp.int32),
            pltpu.VMEM((CBLK, B), jnp.float32),
            pltpu.VMEM((NCH, B), jnp.float32),
            pltpu.VMEM((NCH, B), jnp.int32),
        ],
        compiler_params=pltpu.CompilerParams(
            dimension_semantics=("arbitrary",)),
    )(query, w_dg, keys_pad)


_NC = 2
_NS = 16
_NW = _NC * _NS
_BPW = (B * K) // _NW
_CHUNK = 128
_NCHUNK = _BPW // _CHUNK


def _gather_call(blk_idx, table128):
    mesh = plsc.VectorSubcoreMesh(core_axis_name="c", subcore_axis_name="s")

    @functools.partial(
        pl.kernel,
        mesh=mesh,
        out_type=jax.ShapeDtypeStruct((B * K, 2 * D), jnp.float32),
        scratch_types=[
            pltpu.VMEM((_NCHUNK, _CHUNK), jnp.int32),
            pltpu.VMEM((2, _CHUNK, 2 * D), jnp.float32),
            pltpu.SemaphoreType.DMA,
        ],
    )
    def _gather(idx_hbm, table_hbm, out_hbm, idx_v, bufs, sem):
        wid = lax.axis_index("s") * _NC + lax.axis_index("c")
        base = wid * _BPW
        pltpu.sync_copy(idx_hbm.at[pl.ds(wid * _NCHUNK, _NCHUNK)], idx_v)
        handles = []
        for j in range(_NCHUNK):
            handles.append(
                pltpu.async_copy(table_hbm.at[idx_v.at[j]],
                                 bufs.at[j % 2], sem))
            if j > 0:
                handles[j - 1].wait()
                pltpu.sync_copy(
                    bufs.at[(j - 1) % 2],
                    out_hbm.at[pl.ds(base + (j - 1) * _CHUNK, _CHUNK)])
        handles[-1].wait()
        pltpu.sync_copy(
            bufs.at[(_NCHUNK - 1) % 2],
            out_hbm.at[pl.ds(base + (_NCHUNK - 1) * _CHUNK, _CHUNK)])

    return _gather(blk_idx, table128)


def _ca1_body(g_ref, h_ref, w_ref, o_ref):
    g = g_ref[...]
    h = h_ref[...]
    lo = g[:, :D]
    hi = g[:, D:]
    rows = jnp.where(h == 0, lo, hi)
    o_ref[...] = lax.dot_general(rows, w_ref[...],
                                 (((1,), (1,)), ((), ())),
                                 preferred_element_type=jnp.float32)


def _ca1_call(g, half, w_ca1):
    rows = B * K
    blk = 8192
    return pl.pallas_call(
        _ca1_body,
        grid=(rows // blk,),
        in_specs=[
            pl.BlockSpec((blk, 2 * D), lambda i: (i, 0)),
            pl.BlockSpec((blk, 1), lambda i: (i, 0)),
            pl.BlockSpec((D, D), lambda i: (0, 0)),
        ],
        out_specs=pl.BlockSpec((blk, D), lambda i: (i, 0)),
        out_shape=jax.ShapeDtypeStruct((rows, D), jnp.float32),
    )(g, half, w_ca1)


def kernel(query, keys_mem, values_mem, W_dg, W_ca1):
    blk, half, nov = _topk_call(query, keys_mem, W_dg)
    table128 = values_mem.reshape(CAP // 2, 2 * D)
    gathered = _gather_call(blk.reshape(-1, _CHUNK), table128)
    recalled = _ca1_call(gathered, half.reshape(-1, 1), W_ca1)
    return recalled.reshape(B, K, D), nov.reshape(B)
